# baseline (speedup 1.0000x reference)
import functools
import math

import jax
import jax.numpy as jnp
from jax.experimental import pallas as pl
from jax.experimental.pallas import tpu as pltpu

_NEG = -1e30


def _b16(x):
    return x.astype(jnp.bfloat16).astype(jnp.float32)


def _bdot(a, b):
    return jnp.dot(a.astype(jnp.bfloat16), b.astype(jnp.bfloat16),
                   preferred_element_type=jnp.float32)


def _bdot_t(a, b):
    return jax.lax.dot_general(
        a.astype(jnp.bfloat16), b.astype(jnp.bfloat16),
        (((1,), (1,)), ((), ())), preferred_element_type=jnp.float32)


def _pick_rows(nres: int) -> int:
    best = 0
    for r in range(1, min(nres, 2048) + 1):
        if nres % r == 0 and (r % 8 == 0 or best == 0):
            best = r
    return best if best else nres


def _ln(x, g, b, eps=1e-5):
    m = jnp.mean(x, axis=-1, keepdims=True)
    xc = x - m
    v = jnp.mean(xc * xc, axis=-1, keepdims=True)
    return xc * jax.lax.rsqrt(v + eps) * g + b


def _gelu(x):
    return 0.5 * x * (1.0 + jax.lax.erf(x / math.sqrt(2.0)))


def _softmax_pool(q, keys, vals, hid):
    scale = 1.0 / math.sqrt(hid)
    q16 = _b16(q)
    att = [jnp.sum(q16 * _b16(kk), axis=1, keepdims=True) * scale
           for kk in keys]
    mx = att[0]
    for a in att[1:]:
        mx = jnp.maximum(mx, a)
    es = [jnp.exp(a - mx) for a in att]
    den = es[0]
    for ee in es[1:]:
        den = den + ee
    ws = [ee / den for ee in es]
    acc = _b16(ws[0]) * _b16(vals[0])
    for ww, vv in zip(ws[1:], vals[1:]):
        acc = acc + _b16(ww) * _b16(vv)
    return acc


def _head(npatch, layer, hid, x, ap,
          fpg_ref, fpb_ref, fpw_ref, fpbias_ref,
          gdg_ref, gdb_ref, gd1w_ref, gd1b_ref, gd2w_ref, gd2b_ref,
          gpw_ref, gpb_ref, qw_ref, qb_ref, lndg_ref, lndb_ref,
          kdw_ref, kdb_ref, lnpg_ref, lnpb_ref, kpw_ref, kpb_ref,
          l1w_ref, l1b_ref, l2w_ref, l2b_ref, l3w_ref, l3b_ref,
          pr1_ref, pr2_ref, out_ref):
    dtok = _bdot_t(_ln(x, fpg_ref[...], fpb_ref[...]),
                   fpw_ref[...]) + fpbias_ref[...]
    d_toks, kd = [], []
    for n in range(npatch):
        dn = _ln(dtok[:, n * hid:(n + 1) * hid], lndg_ref[...], lndb_ref[...])
        d_toks.append(dn)
        kd.append(_bdot_t(dn, kdw_ref[...]) + kdb_ref[...])
    g = _bdot_t(_ln(x, gdg_ref[...], gdb_ref[...]),
                gd1w_ref[...]) + gd1b_ref[...]
    g_d = _bdot_t(_gelu(g), gd2w_ref[...]) + gd2b_ref[...]
    ap_mean = ap[0]
    for l in range(1, layer):
        ap_mean = ap_mean + ap[l]
    ap_mean = ap_mean * (1.0 / layer)
    g_p = _gelu(_bdot_t(ap_mean, gpw_ref[...]) + gpb_ref[...])
    q_all = (_bdot_t(g_d, qw_ref[:, :hid]) + _bdot_t(g_p, qw_ref[:, hid:])
             + qb_ref[...])
    pts, kp = [], []
    for l in range(layer):
        pt = _ln(ap[l], lnpg_ref[...], lnpb_ref[...])
        pts.append(pt)
        kp.append(_bdot_t(pt, kpw_ref[...]) + kpb_ref[...])
    vp = _softmax_pool(q_all, kp, pts, hid)
    vd = _softmax_pool(q_all, kd, d_toks, hid)
    z = (_bdot_t(vd, l1w_ref[:, :hid]) + _bdot_t(vp, l1w_ref[:, hid:])
         + l1b_ref[...])
    z = jnp.where(z >= 0, z, pr1_ref[...] * z)
    z = _bdot_t(z, l2w_ref[...]) + l2b_ref[...]
    z = jnp.where(z >= 0, z, pr2_ref[...] * z)
    out_ref[...] = jnp.sum(_b16(z) * _b16(l3w_ref[...]), axis=1,
                           keepdims=True) + l3b_ref[...]


def _att_kernel(nblk, layer, hid, npatch,
                pea_ref, peb_ref, seg_ref, w2t_ref, wkcat_ref, bkcat_ref,
                w1blk_ref, x_ref, *head_refs):
    out_ref, m_ref, s_ref, o_ref = head_refs[-4:]
    i = pl.program_id(0)
    bsz = s_ref.shape[0]
    half = pea_ref.shape[1]

    @pl.when(i == 0)
    def _init():
        m_ref[...] = jnp.full(m_ref.shape, _NEG, jnp.float32)
        s_ref[...] = jnp.zeros(s_ref.shape, jnp.float32)
        o_ref[...] = jnp.zeros(o_ref.shape, jnp.float32)

    prot16 = jnp.maximum(
        jnp.dot(pea_ref[...].astype(jnp.bfloat16), w2t_ref[:half],
                preferred_element_type=jnp.float32)
        + jnp.dot(peb_ref[...].astype(jnp.bfloat16), w2t_ref[half:],
                  preferred_element_type=jnp.float32),
        0.0).astype(jnp.bfloat16)
    rows = prot16.shape[0]
    seg_row = seg_ref[0]

    kall = jnp.maximum(
        jnp.dot(prot16, wkcat_ref[...], preferred_element_type=jnp.float32)
        .astype(jnp.bfloat16) + bkcat_ref[...],
        jnp.bfloat16(0.0))
    t_all = jnp.dot(kall, w1blk_ref[...],
                    preferred_element_type=jnp.float32)
    m_old = m_ref[...]
    m_new = jnp.maximum(m_old, jnp.max(t_all, axis=0, keepdims=True))
    alpha = _b16(jnp.exp(m_old - m_new))
    e16 = jnp.exp(t_all - m_new).astype(jnp.bfloat16)
    e16t = jnp.transpose(e16)

    @pl.when(jnp.any(m_new > m_old))
    def _rescale():
        s_ref[...] = alpha * s_ref[...]
        o_ref[...] = jnp.repeat(alpha, hid, axis=1) * o_ref[...]
    m_ref[...] = m_new

    lo = jnp.min(seg_row)
    hi = jnp.max(seg_row)
    sb = 32 if bsz % 32 == 0 else bsz
    for j in range(bsz // sb):
        base = j * sb

        @pl.when((hi >= base) & (lo < base + sb))
        def _band(base=base):
            oh = (seg_row - base == jax.lax.broadcasted_iota(
                jnp.int32, (sb, rows), 0)).astype(jnp.bfloat16)
            s_ref[base:base + sb, :] = (
                s_ref[base:base + sb, :] + jax.lax.dot_general(
                    oh, e16, (((1,), (0,)), ((), ())),
                    preferred_element_type=jnp.float32))
            o_band = jnp.concatenate(
                [jax.lax.dot_general(
                    oh * e16t[l:l + 1, :], kall[:, l * hid:(l + 1) * hid],
                    (((1,), (0,)), ((), ())),
                    preferred_element_type=jnp.float32)
                 for l in range(layer)], axis=1)
            o_ref[base:base + sb, :] = o_ref[base:base + sb, :] + o_band

    @pl.when(i == nblk - 1)
    def _fin():
        ap = [o_ref[:, l * hid:(l + 1) * hid]
              / (s_ref[:, l:l + 1] + 1e-16) for l in range(layer)]
        _head(npatch, layer, hid, x_ref[...], ap, *head_refs[:-3])


def kernel(x, pro_emb, params, pro_emb_batch):
    p = params
    bsz, mol_in = x.shape
    nres, pdim = pro_emb.shape
    hid = p['prej2.W'].shape[0]
    layer = len([k for k in p if k.startswith('att') and k.endswith('.q.W')])
    npatch = p['ccfm.fp_proj.W'].shape[0] // hid
    rows = _pick_rows(nres)
    nblk = nres // rows

    w2t = p['prej2.W'].T.astype(jnp.bfloat16)
    wkcat = jnp.concatenate([p['att%d.k.W' % l].T for l in range(layer)],
                            axis=1).astype(jnp.bfloat16)
    bkcat = jnp.concatenate([p['att%d.k.b' % l] for l in range(layer)]
                            )[None, :].astype(jnp.bfloat16)
    w1cat = jnp.concatenate([p['att%d.merge.W' % l][0, :hid]
                             for l in range(layer)])
    lheye = jnp.repeat(jnp.eye(layer, dtype=jnp.float32), hid, axis=0)
    w1blk = (lheye * w1cat[:, None]).astype(jnp.bfloat16)
    seg3 = pro_emb_batch.reshape(nblk, 1, rows)

    head_params = [
        p['ccfm.fp_ln.g'][None, :], p['ccfm.fp_ln.b'][None, :],
        p['ccfm.fp_proj.W'], p['ccfm.fp_proj.b'][None, :],
        p['ccfm.gd_ln.g'][None, :], p['ccfm.gd_ln.b'][None, :],
        p['ccfm.gd1.W'], p['ccfm.gd1.b'][None, :],
        p['ccfm.gd2.W'], p['ccfm.gd2.b'][None, :],
        p['ccfm.gp.W'], p['ccfm.gp.b'][None, :],
        p['ccfm.q.W'], p['ccfm.q.b'][None, :],
        p['ccfm.ln_d.g'][None, :], p['ccfm.ln_d.b'][None, :],
        p['ccfm.k_d.W'], p['ccfm.k_d.b'][None, :],
        p['ccfm.ln_p.g'][None, :], p['ccfm.ln_p.b'][None, :],
        p['ccfm.k_p.W'], p['ccfm.k_p.b'][None, :],
        p['out.l1.W'], p['out.l1.b'][None, :],
        p['out.l2.W'], p['out.l2.b'][None, :],
        p['out.l3.W'], p['out.l3.b'][None, :],
        p['out.prelu1'][None, :], p['out.prelu2'][None, :],
    ]

    def _const2(shape):
        return pl.BlockSpec(shape, lambda i: (0, 0))

    out = pl.pallas_call(
        functools.partial(_att_kernel, nblk, layer, hid, npatch),
        grid=(nblk,),
        in_specs=[
            pl.BlockSpec((rows, pdim // 2), lambda i: (i, 0)),
            pl.BlockSpec((rows, pdim // 2), lambda i: (i, 1)),
            pl.BlockSpec((1, 1, rows), lambda i: (i, 0, 0)),
            _const2((pdim, hid)),
            _const2((hid, layer * hid)),
            _const2((1, layer * hid)),
            _const2((layer * hid, layer)),
            _const2((bsz, mol_in)),
        ] + [_const2(hp.shape) for hp in head_params],
        out_specs=pl.BlockSpec((bsz, 1), lambda i: (0, 0)),
        out_shape=jax.ShapeDtypeStruct((bsz, 1), jnp.float32),
        scratch_shapes=[
            pltpu.VMEM((1, layer), jnp.float32),
            pltpu.VMEM((bsz, layer), jnp.float32),
            pltpu.VMEM((bsz, layer * hid), jnp.float32),
        ],
        compiler_params=pltpu.CompilerParams(
            dimension_semantics=("arbitrary",)),
    )(pro_emb, pro_emb, seg3, w2t, wkcat, bkcat, w1blk, x, *head_params)
    return out[:, 0]


# speedup vs baseline: 1.0593x; 1.0593x over previous
import functools
import math

import jax
import jax.numpy as jnp
from jax.experimental import pallas as pl
from jax.experimental.pallas import tpu as pltpu

_NEG = -1e30


def _b16(x):
    return x.astype(jnp.bfloat16).astype(jnp.float32)


def _bdot(a, b):
    return jnp.dot(a.astype(jnp.bfloat16), b.astype(jnp.bfloat16),
                   preferred_element_type=jnp.float32)


def _bdot_t(a, b):
    return jax.lax.dot_general(
        a.astype(jnp.bfloat16), b.astype(jnp.bfloat16),
        (((1,), (1,)), ((), ())), preferred_element_type=jnp.float32)


def _pick_rows(nres: int) -> int:
    best = 0
    for r in range(1, min(nres, 2048) + 1):
        if nres % r == 0 and (r % 8 == 0 or best == 0):
            best = r
    return best if best else nres


def _ln(x, g, b, eps=1e-5):
    m = jnp.mean(x, axis=-1, keepdims=True)
    xc = x - m
    v = jnp.mean(xc * xc, axis=-1, keepdims=True)
    return xc * jax.lax.rsqrt(v + eps) * g + b


def _gelu(x):
    return 0.5 * x * (1.0 + jax.lax.erf(x / math.sqrt(2.0)))


def _softmax_pool(q, keys, vals, hid):
    scale = 1.0 / math.sqrt(hid)
    q16 = _b16(q)
    att = [jnp.sum(q16 * _b16(kk), axis=1, keepdims=True) * scale
           for kk in keys]
    mx = att[0]
    for a in att[1:]:
        mx = jnp.maximum(mx, a)
    es = [jnp.exp(a - mx) for a in att]
    den = es[0]
    for ee in es[1:]:
        den = den + ee
    ws = [ee / den for ee in es]
    acc = _b16(ws[0]) * _b16(vals[0])
    for ww, vv in zip(ws[1:], vals[1:]):
        acc = acc + _b16(ww) * _b16(vv)
    return acc


def _head(npatch, layer, hid, x, ap,
          fpg_ref, fpb_ref, fpw_ref, fpbias_ref,
          gdg_ref, gdb_ref, gd1w_ref, gd1b_ref, gd2w_ref, gd2b_ref,
          gpw_ref, gpb_ref, qw_ref, qb_ref, lndg_ref, lndb_ref,
          kdw_ref, kdb_ref, lnpg_ref, lnpb_ref, kpw_ref, kpb_ref,
          l1w_ref, l1b_ref, l2w_ref, l2b_ref, l3w_ref, l3b_ref,
          pr1_ref, pr2_ref, out_ref):
    dtok = _bdot_t(_ln(x, fpg_ref[...], fpb_ref[...]),
                   fpw_ref[...]) + fpbias_ref[...]
    d_toks, kd = [], []
    for n in range(npatch):
        dn = _ln(dtok[:, n * hid:(n + 1) * hid], lndg_ref[...], lndb_ref[...])
        d_toks.append(dn)
        kd.append(_bdot_t(dn, kdw_ref[...]) + kdb_ref[...])
    g = _bdot_t(_ln(x, gdg_ref[...], gdb_ref[...]),
                gd1w_ref[...]) + gd1b_ref[...]
    g_d = _bdot_t(_gelu(g), gd2w_ref[...]) + gd2b_ref[...]
    ap_mean = ap[0]
    for l in range(1, layer):
        ap_mean = ap_mean + ap[l]
    ap_mean = ap_mean * (1.0 / layer)
    g_p = _gelu(_bdot_t(ap_mean, gpw_ref[...]) + gpb_ref[...])
    q_all = (_bdot_t(g_d, qw_ref[:, :hid]) + _bdot_t(g_p, qw_ref[:, hid:])
             + qb_ref[...])
    pts, kp = [], []
    for l in range(layer):
        pt = _ln(ap[l], lnpg_ref[...], lnpb_ref[...])
        pts.append(pt)
        kp.append(_bdot_t(pt, kpw_ref[...]) + kpb_ref[...])
    vp = _softmax_pool(q_all, kp, pts, hid)
    vd = _softmax_pool(q_all, kd, d_toks, hid)
    z = (_bdot_t(vd, l1w_ref[:, :hid]) + _bdot_t(vp, l1w_ref[:, hid:])
         + l1b_ref[...])
    z = jnp.where(z >= 0, z, pr1_ref[...] * z)
    z = _bdot_t(z, l2w_ref[...]) + l2b_ref[...]
    z = jnp.where(z >= 0, z, pr2_ref[...] * z)
    out_ref[...] = jnp.sum(_b16(z) * _b16(l3w_ref[...]), axis=1,
                           keepdims=True) + l3b_ref[...]


def _att_kernel(nblk, layer, hid, npatch,
                pe_ref, seg_ref, w2_ref, wk0_ref, bkcat_ref,
                w1row_ref, x_ref, *head_refs):
    out_ref, m_ref, s_ref, o_ref, w2t_ref, wkcat_ref, w1blk_ref = \
        head_refs[-7:]
    i = pl.program_id(0)
    bsz = s_ref.shape[0]
    lh = layer * hid

    @pl.when(i == 0)
    def _init():
        m_ref[...] = jnp.full(m_ref.shape, _NEG, jnp.float32)
        s_ref[...] = jnp.zeros(s_ref.shape, jnp.float32)
        o_ref[...] = jnp.zeros(o_ref.shape, jnp.float32)
        w2t_ref[...] = jnp.transpose(w2_ref[...]).astype(jnp.bfloat16)
        wkcat_ref[...] = jnp.transpose(wk0_ref[...]).astype(jnp.bfloat16)
        w1col = jnp.transpose(w1row_ref[...])
        blkmask = (jax.lax.broadcasted_iota(jnp.int32, (lh, layer), 0) // hid
                   == jax.lax.broadcasted_iota(jnp.int32, (lh, layer), 1))
        w1blk_ref[...] = jnp.where(blkmask, w1col, 0.0).astype(jnp.bfloat16)

    prot16 = jnp.maximum(
        jnp.dot(pe_ref[...].astype(jnp.bfloat16), w2t_ref[...],
                preferred_element_type=jnp.float32),
        0.0).astype(jnp.bfloat16)
    rows = prot16.shape[0]
    seg_row = seg_ref[0]

    kall = jnp.maximum(
        jnp.dot(prot16, wkcat_ref[...], preferred_element_type=jnp.float32)
        .astype(jnp.bfloat16) + bkcat_ref[...].astype(jnp.bfloat16),
        jnp.bfloat16(0.0))
    t_all = jnp.dot(kall, w1blk_ref[...],
                    preferred_element_type=jnp.float32)
    m_old = m_ref[...]
    m_new = jnp.maximum(m_old, jnp.max(t_all, axis=0, keepdims=True))
    alpha = _b16(jnp.exp(m_old - m_new))
    e16 = jnp.exp(t_all - m_new).astype(jnp.bfloat16)
    e16t = jnp.transpose(e16)

    @pl.when(jnp.any(m_new > m_old))
    def _rescale():
        s_ref[...] = alpha * s_ref[...]
        o_ref[...] = jnp.repeat(alpha, hid, axis=1) * o_ref[...]
    m_ref[...] = m_new

    lo = jnp.min(seg_row)
    hi = jnp.max(seg_row)
    sb = 32 if bsz % 32 == 0 else bsz
    for j in range(bsz // sb):
        base = j * sb

        @pl.when((hi >= base) & (lo < base + sb))
        def _band(base=base):
            oh = (seg_row - base == jax.lax.broadcasted_iota(
                jnp.int32, (sb, rows), 0)).astype(jnp.bfloat16)
            s_ref[base:base + sb, :] = (
                s_ref[base:base + sb, :] + jax.lax.dot_general(
                    oh, e16, (((1,), (0,)), ((), ())),
                    preferred_element_type=jnp.float32))
            o_band = jnp.concatenate(
                [jax.lax.dot_general(
                    oh * e16t[l:l + 1, :], kall[:, l * hid:(l + 1) * hid],
                    (((1,), (0,)), ((), ())),
                    preferred_element_type=jnp.float32)
                 for l in range(layer)], axis=1)
            o_ref[base:base + sb, :] = o_ref[base:base + sb, :] + o_band

    @pl.when(i == nblk - 1)
    def _fin():
        ap = [o_ref[:, l * hid:(l + 1) * hid]
              / (s_ref[:, l:l + 1] + 1e-16) for l in range(layer)]
        _head(npatch, layer, hid, x_ref[...], ap, *head_refs[:-6])


def kernel(x, pro_emb, params, pro_emb_batch):
    p = params
    bsz, mol_in = x.shape
    nres, pdim = pro_emb.shape
    hid = p['prej2.W'].shape[0]
    layer = len([k for k in p if k.startswith('att') and k.endswith('.q.W')])
    npatch = p['ccfm.fp_proj.W'].shape[0] // hid
    rows = _pick_rows(nres)
    nblk = nres // rows

    wk0 = jnp.concatenate([p['att%d.k.W' % l] for l in range(layer)],
                          axis=0)
    bkcat = jnp.concatenate([p['att%d.k.b' % l] for l in range(layer)]
                            )[None, :]
    w1row = jnp.concatenate([p['att%d.merge.W' % l][:, :hid]
                             for l in range(layer)], axis=1)
    seg3 = pro_emb_batch.reshape(nblk, 1, rows)

    head_params = [
        p['ccfm.fp_ln.g'][None, :], p['ccfm.fp_ln.b'][None, :],
        p['ccfm.fp_proj.W'], p['ccfm.fp_proj.b'][None, :],
        p['ccfm.gd_ln.g'][None, :], p['ccfm.gd_ln.b'][None, :],
        p['ccfm.gd1.W'], p['ccfm.gd1.b'][None, :],
        p['ccfm.gd2.W'], p['ccfm.gd2.b'][None, :],
        p['ccfm.gp.W'], p['ccfm.gp.b'][None, :],
        p['ccfm.q.W'], p['ccfm.q.b'][None, :],
        p['ccfm.ln_d.g'][None, :], p['ccfm.ln_d.b'][None, :],
        p['ccfm.k_d.W'], p['ccfm.k_d.b'][None, :],
        p['ccfm.ln_p.g'][None, :], p['ccfm.ln_p.b'][None, :],
        p['ccfm.k_p.W'], p['ccfm.k_p.b'][None, :],
        p['out.l1.W'], p['out.l1.b'][None, :],
        p['out.l2.W'], p['out.l2.b'][None, :],
        p['out.l3.W'], p['out.l3.b'][None, :],
        p['out.prelu1'][None, :], p['out.prelu2'][None, :],
    ]

    def _const2(shape):
        return pl.BlockSpec(shape, lambda i: (0, 0))

    out = pl.pallas_call(
        functools.partial(_att_kernel, nblk, layer, hid, npatch),
        grid=(nblk,),
        in_specs=[
            pl.BlockSpec((rows, pdim), lambda i: (i, 0)),
            pl.BlockSpec((1, 1, rows), lambda i: (i, 0, 0)),
            _const2((hid, pdim)),
            _const2((layer * hid, hid)),
            _const2((1, layer * hid)),
            _const2((1, layer * hid)),
            _const2((bsz, mol_in)),
        ] + [_const2(hp.shape) for hp in head_params],
        out_specs=pl.BlockSpec((bsz, 1), lambda i: (0, 0)),
        out_shape=jax.ShapeDtypeStruct((bsz, 1), jnp.float32),
        scratch_shapes=[
            pltpu.VMEM((1, layer), jnp.float32),
            pltpu.VMEM((bsz, layer), jnp.float32),
            pltpu.VMEM((bsz, layer * hid), jnp.float32),
            pltpu.VMEM((pdim, hid), jnp.bfloat16),
            pltpu.VMEM((hid, layer * hid), jnp.bfloat16),
            pltpu.VMEM((layer * hid, layer), jnp.bfloat16),
        ],
        compiler_params=pltpu.CompilerParams(
            dimension_semantics=("arbitrary",)),
    )(pro_emb, seg3, p['prej2.W'], wk0, bkcat, w1row, x, *head_params)
    return out[:, 0]


# speedup vs baseline: 1.1578x; 1.0930x over previous
import functools
import math

import jax
import jax.numpy as jnp
from jax.experimental import pallas as pl
from jax.experimental.pallas import tpu as pltpu

_NEG = -1e30


def _b16(x):
    return x.astype(jnp.bfloat16).astype(jnp.float32)


def _bdot(a, b):
    return jnp.dot(a.astype(jnp.bfloat16), b.astype(jnp.bfloat16),
                   preferred_element_type=jnp.float32)


def _bdot_t(a, b):
    return jax.lax.dot_general(
        a.astype(jnp.bfloat16), b.astype(jnp.bfloat16),
        (((1,), (1,)), ((), ())), preferred_element_type=jnp.float32)


def _pick_rows(nres: int) -> int:
    best = 0
    for r in range(1, min(nres, 2048) + 1):
        if nres % r == 0 and (r % 8 == 0 or best == 0):
            best = r
    return best if best else nres


def _ln(x, g, b, eps=1e-5):
    m = jnp.mean(x, axis=-1, keepdims=True)
    xc = x - m
    v = jnp.mean(xc * xc, axis=-1, keepdims=True)
    return xc * jax.lax.rsqrt(v + eps) * g + b


def _gelu(x):
    return 0.5 * x * (1.0 + jax.lax.erf(x / math.sqrt(2.0)))


def _softmax_pool(q, keys, vals, hid):
    scale = 1.0 / math.sqrt(hid)
    q16 = _b16(q)
    att = [jnp.sum(q16 * _b16(kk), axis=1, keepdims=True) * scale
           for kk in keys]
    mx = att[0]
    for a in att[1:]:
        mx = jnp.maximum(mx, a)
    es = [jnp.exp(a - mx) for a in att]
    den = es[0]
    for ee in es[1:]:
        den = den + ee
    ws = [ee / den for ee in es]
    acc = _b16(ws[0]) * _b16(vals[0])
    for ww, vv in zip(ws[1:], vals[1:]):
        acc = acc + _b16(ww) * _b16(vv)
    return acc


def _head(npatch, layer, hid, x, ap,
          fpg_ref, fpb_ref, fpw_ref, fpbias_ref,
          gdg_ref, gdb_ref, gd1w_ref, gd1b_ref, gd2w_ref, gd2b_ref,
          gpw_ref, gpb_ref, qw_ref, qb_ref, lndg_ref, lndb_ref,
          kdw_ref, kdb_ref, lnpg_ref, lnpb_ref, kpw_ref, kpb_ref,
          l1w_ref, l1b_ref, l2w_ref, l2b_ref, l3w_ref, l3b_ref,
          pr1_ref, pr2_ref, out_ref):
    dtok = _bdot_t(_ln(x, fpg_ref[...], fpb_ref[...]),
                   fpw_ref[...]) + fpbias_ref[...]
    d_toks, kd = [], []
    for n in range(npatch):
        dn = _ln(dtok[:, n * hid:(n + 1) * hid], lndg_ref[...], lndb_ref[...])
        d_toks.append(dn)
        kd.append(_bdot_t(dn, kdw_ref[...]) + kdb_ref[...])
    g = _bdot_t(_ln(x, gdg_ref[...], gdb_ref[...]),
                gd1w_ref[...]) + gd1b_ref[...]
    g_d = _bdot_t(_gelu(g), gd2w_ref[...]) + gd2b_ref[...]
    ap_mean = ap[0]
    for l in range(1, layer):
        ap_mean = ap_mean + ap[l]
    ap_mean = ap_mean * (1.0 / layer)
    g_p = _gelu(_bdot_t(ap_mean, gpw_ref[...]) + gpb_ref[...])
    q_all = (_bdot_t(g_d, qw_ref[:, :hid]) + _bdot_t(g_p, qw_ref[:, hid:])
             + qb_ref[...])
    pts, kp = [], []
    for l in range(layer):
        pt = _ln(ap[l], lnpg_ref[...], lnpb_ref[...])
        pts.append(pt)
        kp.append(_bdot_t(pt, kpw_ref[...]) + kpb_ref[...])
    vp = _softmax_pool(q_all, kp, pts, hid)
    vd = _softmax_pool(q_all, kd, d_toks, hid)
    z = (_bdot_t(vd, l1w_ref[:, :hid]) + _bdot_t(vp, l1w_ref[:, hid:])
         + l1b_ref[...])
    z = jnp.where(z >= 0, z, pr1_ref[...] * z)
    z = _bdot_t(z, l2w_ref[...]) + l2b_ref[...]
    z = jnp.where(z >= 0, z, pr2_ref[...] * z)
    out_ref[...] = jnp.sum(_b16(z) * _b16(l3w_ref[...]), axis=1,
                           keepdims=True) + l3b_ref[...]


def _att_kernel(nblk, layer, hid, npatch, *refs):
    pe_ref, seg_ref, w2_ref = refs[:3]
    wk_refs = refs[3:3 + layer]
    bk_refs = refs[3 + layer:3 + 2 * layer]
    mg_refs = refs[3 + 2 * layer:3 + 3 * layer]
    x_ref = refs[3 + 3 * layer]
    head_param_refs = refs[4 + 3 * layer:-8]
    out_ref = refs[-8]
    (m_ref, s_ref, o_ref, w2t_ref, wkcat_ref, bkcat_ref,
     w1blk_ref) = refs[-7:]
    i = pl.program_id(0)
    bsz = s_ref.shape[0]

    @pl.when(i == 0)
    def _init():
        m_ref[...] = jnp.full(m_ref.shape, _NEG, jnp.float32)
        s_ref[...] = jnp.zeros(s_ref.shape, jnp.float32)
        o_ref[...] = jnp.zeros(o_ref.shape, jnp.float32)
        w2t_ref[...] = jnp.transpose(w2_ref[...]).astype(jnp.bfloat16)
        w1blk_ref[...] = jnp.zeros(w1blk_ref.shape, jnp.bfloat16)
        for l in range(layer):
            wkcat_ref[:, l * hid:(l + 1) * hid] = jnp.transpose(
                wk_refs[l][...]).astype(jnp.bfloat16)
            bkcat_ref[:, l * hid:(l + 1) * hid] = (
                bk_refs[l][...].astype(jnp.bfloat16))
            w1blk_ref[l * hid:(l + 1) * hid, l:l + 1] = jnp.transpose(
                mg_refs[l][:, :hid]).astype(jnp.bfloat16)

    prot16 = jnp.maximum(
        jnp.dot(pe_ref[...].astype(jnp.bfloat16), w2t_ref[...],
                preferred_element_type=jnp.float32),
        0.0).astype(jnp.bfloat16)
    rows = prot16.shape[0]
    seg_row = seg_ref[0]

    kall = jnp.maximum(
        jnp.dot(prot16, wkcat_ref[...], preferred_element_type=jnp.float32)
        .astype(jnp.bfloat16) + bkcat_ref[...],
        jnp.bfloat16(0.0))
    t_all = jnp.dot(kall, w1blk_ref[...],
                    preferred_element_type=jnp.float32)
    m_old = m_ref[...]
    m_new = jnp.maximum(m_old, jnp.max(t_all, axis=0, keepdims=True))
    alpha = _b16(jnp.exp(m_old - m_new))
    e16 = jnp.exp(t_all - m_new).astype(jnp.bfloat16)
    e16t = jnp.transpose(e16)

    @pl.when(jnp.any(m_new > m_old))
    def _rescale():
        s_ref[...] = alpha * s_ref[...]
        o_ref[...] = jnp.repeat(alpha, hid, axis=1) * o_ref[...]
    m_ref[...] = m_new

    lo = jnp.min(seg_row)
    hi = jnp.max(seg_row)
    sb = 32 if bsz % 32 == 0 else bsz
    for j in range(bsz // sb):
        base = j * sb

        @pl.when((hi >= base) & (lo < base + sb))
        def _band(base=base):
            oh = (seg_row - base == jax.lax.broadcasted_iota(
                jnp.int32, (sb, rows), 0)).astype(jnp.bfloat16)
            s_ref[base:base + sb, :] = (
                s_ref[base:base + sb, :] + jax.lax.dot_general(
                    oh, e16, (((1,), (0,)), ((), ())),
                    preferred_element_type=jnp.float32))
            o_band = jnp.concatenate(
                [jax.lax.dot_general(
                    oh * e16t[l:l + 1, :], kall[:, l * hid:(l + 1) * hid],
                    (((1,), (0,)), ((), ())),
                    preferred_element_type=jnp.float32)
                 for l in range(layer)], axis=1)
            o_ref[base:base + sb, :] = o_ref[base:base + sb, :] + o_band

    @pl.when(i == nblk - 1)
    def _fin():
        ap = [o_ref[:, l * hid:(l + 1) * hid]
              / (s_ref[:, l:l + 1] + 1e-16) for l in range(layer)]
        _head(npatch, layer, hid, x_ref[...], ap, *head_param_refs, out_ref)


def kernel(x, pro_emb, params, pro_emb_batch):
    p = params
    bsz, mol_in = x.shape
    nres, pdim = pro_emb.shape
    hid = p['prej2.W'].shape[0]
    layer = len([k for k in p if k.startswith('att') and k.endswith('.q.W')])
    npatch = p['ccfm.fp_proj.W'].shape[0] // hid
    rows = _pick_rows(nres)
    nblk = nres // rows

    att_params = ([p['att%d.k.W' % l] for l in range(layer)]
                  + [p['att%d.k.b' % l][None, :] for l in range(layer)]
                  + [p['att%d.merge.W' % l] for l in range(layer)])
    seg3 = pro_emb_batch.reshape(nblk, 1, rows)

    head_params = [
        p['ccfm.fp_ln.g'][None, :], p['ccfm.fp_ln.b'][None, :],
        p['ccfm.fp_proj.W'], p['ccfm.fp_proj.b'][None, :],
        p['ccfm.gd_ln.g'][None, :], p['ccfm.gd_ln.b'][None, :],
        p['ccfm.gd1.W'], p['ccfm.gd1.b'][None, :],
        p['ccfm.gd2.W'], p['ccfm.gd2.b'][None, :],
        p['ccfm.gp.W'], p['ccfm.gp.b'][None, :],
        p['ccfm.q.W'], p['ccfm.q.b'][None, :],
        p['ccfm.ln_d.g'][None, :], p['ccfm.ln_d.b'][None, :],
        p['ccfm.k_d.W'], p['ccfm.k_d.b'][None, :],
        p['ccfm.ln_p.g'][None, :], p['ccfm.ln_p.b'][None, :],
        p['ccfm.k_p.W'], p['ccfm.k_p.b'][None, :],
        p['out.l1.W'], p['out.l1.b'][None, :],
        p['out.l2.W'], p['out.l2.b'][None, :],
        p['out.l3.W'], p['out.l3.b'][None, :],
        p['out.prelu1'][None, :], p['out.prelu2'][None, :],
    ]

    def _const2(shape):
        return pl.BlockSpec(shape, lambda i: (0, 0))

    out = pl.pallas_call(
        functools.partial(_att_kernel, nblk, layer, hid, npatch),
        grid=(nblk,),
        in_specs=[
            pl.BlockSpec((rows, pdim), lambda i: (i, 0)),
            pl.BlockSpec((1, 1, rows), lambda i: (i, 0, 0)),
            _const2((hid, pdim)),
        ] + [_const2(ap_.shape) for ap_ in att_params]
        + [_const2((bsz, mol_in))]
        + [_const2(hp.shape) for hp in head_params],
        out_specs=pl.BlockSpec((bsz, 1), lambda i: (0, 0)),
        out_shape=jax.ShapeDtypeStruct((bsz, 1), jnp.float32),
        scratch_shapes=[
            pltpu.VMEM((1, layer), jnp.float32),
            pltpu.VMEM((bsz, layer), jnp.float32),
            pltpu.VMEM((bsz, layer * hid), jnp.float32),
            pltpu.VMEM((pdim, hid), jnp.bfloat16),
            pltpu.VMEM((hid, layer * hid), jnp.bfloat16),
            pltpu.VMEM((1, layer * hid), jnp.bfloat16),
            pltpu.VMEM((layer * hid, layer), jnp.bfloat16),
        ],
        compiler_params=pltpu.CompilerParams(
            dimension_semantics=("arbitrary",)),
    )(pro_emb, seg3, p['prej2.W'], *att_params, x, *head_params)
    return out[:, 0]


# speedup vs baseline: 1.1650x; 1.0062x over previous
import functools
import math

import jax
import jax.numpy as jnp
from jax.experimental import pallas as pl
from jax.experimental.pallas import tpu as pltpu

_NEG = -1e30


def _b16(x):
    return x.astype(jnp.bfloat16).astype(jnp.float32)


def _bdot(a, b):
    return jnp.dot(a.astype(jnp.bfloat16), b.astype(jnp.bfloat16),
                   preferred_element_type=jnp.float32)


def _bdot_t(a, b):
    return jax.lax.dot_general(
        a.astype(jnp.bfloat16), b.astype(jnp.bfloat16),
        (((1,), (1,)), ((), ())), preferred_element_type=jnp.float32)


def _pick_rows(nres: int) -> int:
    best = 0
    for r in range(1, min(nres, 2048) + 1):
        if nres % r == 0 and (r % 8 == 0 or best == 0):
            best = r
    return best if best else nres


def _ln(x, g, b, eps=1e-5):
    m = jnp.mean(x, axis=-1, keepdims=True)
    xc = x - m
    v = jnp.mean(xc * xc, axis=-1, keepdims=True)
    return xc * jax.lax.rsqrt(v + eps) * g + b


def _gelu(x):
    return 0.5 * x * (1.0 + jax.lax.erf(x / math.sqrt(2.0)))


def _softmax_pool(q, keys, vals, hid):
    scale = 1.0 / math.sqrt(hid)
    q16 = _b16(q)
    att = [jnp.sum(q16 * _b16(kk), axis=1, keepdims=True) * scale
           for kk in keys]
    mx = att[0]
    for a in att[1:]:
        mx = jnp.maximum(mx, a)
    es = [jnp.exp(a - mx) for a in att]
    den = es[0]
    for ee in es[1:]:
        den = den + ee
    ws = [ee / den for ee in es]
    acc = _b16(ws[0]) * _b16(vals[0])
    for ww, vv in zip(ws[1:], vals[1:]):
        acc = acc + _b16(ww) * _b16(vv)
    return acc


def _head(npatch, layer, hid, x, ap,
          fpg_ref, fpb_ref, fpw_ref, fpbias_ref,
          gdg_ref, gdb_ref, gd1w_ref, gd1b_ref, gd2w_ref, gd2b_ref,
          gpw_ref, gpb_ref, qw_ref, qb_ref, lndg_ref, lndb_ref,
          kdw_ref, kdb_ref, lnpg_ref, lnpb_ref, kpw_ref, kpb_ref,
          l1w_ref, l1b_ref, l2w_ref, l2b_ref, l3w_ref, l3b_ref,
          pr1_ref, pr2_ref, out_ref):
    dtok = _bdot_t(_ln(x, fpg_ref[...], fpb_ref[...]),
                   fpw_ref[...]) + fpbias_ref[...]
    d_toks, kd = [], []
    for n in range(npatch):
        dn = _ln(dtok[:, n * hid:(n + 1) * hid], lndg_ref[...], lndb_ref[...])
        d_toks.append(dn)
        kd.append(_bdot_t(dn, kdw_ref[...]) + kdb_ref[...])
    g = _bdot_t(_ln(x, gdg_ref[...], gdb_ref[...]),
                gd1w_ref[...]) + gd1b_ref[...]
    g_d = _bdot_t(_gelu(g), gd2w_ref[...]) + gd2b_ref[...]
    ap_mean = ap[0]
    for l in range(1, layer):
        ap_mean = ap_mean + ap[l]
    ap_mean = ap_mean * (1.0 / layer)
    g_p = _gelu(_bdot_t(ap_mean, gpw_ref[...]) + gpb_ref[...])
    q_all = (_bdot_t(g_d, qw_ref[:, :hid]) + _bdot_t(g_p, qw_ref[:, hid:])
             + qb_ref[...])
    pts, kp = [], []
    for l in range(layer):
        pt = _ln(ap[l], lnpg_ref[...], lnpb_ref[...])
        pts.append(pt)
        kp.append(_bdot_t(pt, kpw_ref[...]) + kpb_ref[...])
    vp = _softmax_pool(q_all, kp, pts, hid)
    vd = _softmax_pool(q_all, kd, d_toks, hid)
    z = (_bdot_t(vd, l1w_ref[:, :hid]) + _bdot_t(vp, l1w_ref[:, hid:])
         + l1b_ref[...])
    z = jnp.where(z >= 0, z, pr1_ref[...] * z)
    z = _bdot_t(z, l2w_ref[...]) + l2b_ref[...]
    z = jnp.where(z >= 0, z, pr2_ref[...] * z)
    out_ref[...] = (jnp.sum(_b16(z) * _b16(l3w_ref[...]), axis=1,
                            keepdims=True) + l3b_ref[...])[:, 0]


def _att_kernel(nblk, layer, hid, npatch, *refs):
    pe_ref, seg_ref, w2_ref = refs[:3]
    wk_refs = refs[3:3 + layer]
    bk_refs = refs[3 + layer:3 + 2 * layer]
    mg_refs = refs[3 + 2 * layer:3 + 3 * layer]
    x_ref = refs[3 + 3 * layer]
    head_param_refs = refs[4 + 3 * layer:-8]
    out_ref = refs[-8]
    (m_ref, s_ref, o_ref, w2t_ref, wkcat_ref, bkcat_ref,
     w1blk_ref) = refs[-7:]
    i = pl.program_id(0)
    bsz = s_ref.shape[0]

    @pl.when(i == 0)
    def _init():
        m_ref[...] = jnp.full(m_ref.shape, _NEG, jnp.float32)
        s_ref[...] = jnp.zeros(s_ref.shape, jnp.float32)
        o_ref[...] = jnp.zeros(o_ref.shape, jnp.float32)
        w2t_ref[...] = jnp.transpose(w2_ref[...]).astype(jnp.bfloat16)
        w1blk_ref[...] = jnp.zeros(w1blk_ref.shape, jnp.bfloat16)
        for l in range(layer):
            wkcat_ref[:, l * hid:(l + 1) * hid] = jnp.transpose(
                wk_refs[l][...]).astype(jnp.bfloat16)
            bkcat_ref[:, l * hid:(l + 1) * hid] = (
                bk_refs[l][...].astype(jnp.bfloat16))
            w1blk_ref[l * hid:(l + 1) * hid, l:l + 1] = jnp.transpose(
                mg_refs[l][:, :hid]).astype(jnp.bfloat16)

    prot16 = jnp.maximum(
        jnp.dot(pe_ref[...].astype(jnp.bfloat16), w2t_ref[...],
                preferred_element_type=jnp.float32),
        0.0).astype(jnp.bfloat16)
    rows = prot16.shape[0]
    seg_row = seg_ref[0]

    kall = jnp.maximum(
        jnp.dot(prot16, wkcat_ref[...], preferred_element_type=jnp.float32)
        .astype(jnp.bfloat16) + bkcat_ref[...],
        jnp.bfloat16(0.0))
    t_all = jnp.dot(kall, w1blk_ref[...],
                    preferred_element_type=jnp.float32)
    m_old = m_ref[...]
    m_new = jnp.maximum(m_old, jnp.max(t_all, axis=0, keepdims=True))
    alpha = _b16(jnp.exp(m_old - m_new))
    e16 = jnp.exp(t_all - m_new).astype(jnp.bfloat16)
    e16t = jnp.transpose(e16)

    @pl.when(jnp.any(m_new > m_old))
    def _rescale():
        s_ref[...] = alpha * s_ref[...]
        o_ref[...] = jnp.repeat(alpha, hid, axis=1) * o_ref[...]
    m_ref[...] = m_new

    lo = jnp.min(seg_row)
    hi = jnp.max(seg_row)
    sb = 32 if bsz % 32 == 0 else bsz
    for j in range(bsz // sb):
        base = j * sb

        @pl.when((hi >= base) & (lo < base + sb))
        def _band(base=base):
            oh = (seg_row - base == jax.lax.broadcasted_iota(
                jnp.int32, (sb, rows), 0)).astype(jnp.bfloat16)
            s_ref[base:base + sb, :] = (
                s_ref[base:base + sb, :] + jax.lax.dot_general(
                    oh, e16, (((1,), (0,)), ((), ())),
                    preferred_element_type=jnp.float32))
            o_band = jnp.concatenate(
                [jax.lax.dot_general(
                    oh * e16t[l:l + 1, :], kall[:, l * hid:(l + 1) * hid],
                    (((1,), (0,)), ((), ())),
                    preferred_element_type=jnp.float32)
                 for l in range(layer)], axis=1)
            o_ref[base:base + sb, :] = o_ref[base:base + sb, :] + o_band

    @pl.when(i == nblk - 1)
    def _fin():
        ap = [o_ref[:, l * hid:(l + 1) * hid]
              / (s_ref[:, l:l + 1] + 1e-16) for l in range(layer)]
        _head(npatch, layer, hid, x_ref[...], ap, *head_param_refs, out_ref)


def kernel(x, pro_emb, params, pro_emb_batch):
    p = params
    bsz, mol_in = x.shape
    nres, pdim = pro_emb.shape
    hid = p['prej2.W'].shape[0]
    layer = len([k for k in p if k.startswith('att') and k.endswith('.q.W')])
    npatch = p['ccfm.fp_proj.W'].shape[0] // hid
    rows = _pick_rows(nres)
    nblk = nres // rows

    att_params = ([p['att%d.k.W' % l] for l in range(layer)]
                  + [p['att%d.k.b' % l][None, :] for l in range(layer)]
                  + [p['att%d.merge.W' % l] for l in range(layer)])
    seg3 = pro_emb_batch.reshape(nblk, 1, rows)

    head_params = [
        p['ccfm.fp_ln.g'][None, :], p['ccfm.fp_ln.b'][None, :],
        p['ccfm.fp_proj.W'], p['ccfm.fp_proj.b'][None, :],
        p['ccfm.gd_ln.g'][None, :], p['ccfm.gd_ln.b'][None, :],
        p['ccfm.gd1.W'], p['ccfm.gd1.b'][None, :],
        p['ccfm.gd2.W'], p['ccfm.gd2.b'][None, :],
        p['ccfm.gp.W'], p['ccfm.gp.b'][None, :],
        p['ccfm.q.W'], p['ccfm.q.b'][None, :],
        p['ccfm.ln_d.g'][None, :], p['ccfm.ln_d.b'][None, :],
        p['ccfm.k_d.W'], p['ccfm.k_d.b'][None, :],
        p['ccfm.ln_p.g'][None, :], p['ccfm.ln_p.b'][None, :],
        p['ccfm.k_p.W'], p['ccfm.k_p.b'][None, :],
        p['out.l1.W'], p['out.l1.b'][None, :],
        p['out.l2.W'], p['out.l2.b'][None, :],
        p['out.l3.W'], p['out.l3.b'][None, :],
        p['out.prelu1'][None, :], p['out.prelu2'][None, :],
    ]

    def _const2(shape):
        return pl.BlockSpec(shape, lambda i: (0, 0))

    out = pl.pallas_call(
        functools.partial(_att_kernel, nblk, layer, hid, npatch),
        grid=(nblk,),
        in_specs=[
            pl.BlockSpec((rows, pdim), lambda i: (i, 0)),
            pl.BlockSpec((1, 1, rows), lambda i: (i, 0, 0)),
            _const2((hid, pdim)),
        ] + [_const2(ap_.shape) for ap_ in att_params]
        + [_const2((bsz, mol_in))]
        + [_const2(hp.shape) for hp in head_params],
        out_specs=pl.BlockSpec((bsz,), lambda i: (0,)),
        out_shape=jax.ShapeDtypeStruct((bsz,), jnp.float32),
        scratch_shapes=[
            pltpu.VMEM((1, layer), jnp.float32),
            pltpu.VMEM((bsz, layer), jnp.float32),
            pltpu.VMEM((bsz, layer * hid), jnp.float32),
            pltpu.VMEM((pdim, hid), jnp.bfloat16),
            pltpu.VMEM((hid, layer * hid), jnp.bfloat16),
            pltpu.VMEM((1, layer * hid), jnp.bfloat16),
            pltpu.VMEM((layer * hid, layer), jnp.bfloat16),
        ],
        compiler_params=pltpu.CompilerParams(
            dimension_semantics=("arbitrary",)),
    )(pro_emb, seg3, p['prej2.W'], *att_params, x, *head_params)
    return out
